# Initial kernel scaffold; baseline (speedup 1.0000x reference)
import jax
import jax.numpy as jnp
from jax.experimental import pallas as pl


def kernel(pred, iou):
    raise NotImplementedError("write your pallas kernel here")


# speedup vs baseline: 3.9218x; 3.9218x over previous
import jax
import jax.numpy as jnp
from jax.experimental import pallas as pl
from jax.experimental.pallas import tpu as pltpu

_N = 1048576
_ROWS = 8192
_COLS = 128
_BLK_ROWS = 512


def _tc_body(p_ref, i_ref, out_ref, acc_ref):
    step = pl.program_id(0)
    p = p_ref[...]
    iou = i_ref[...]
    pos = iou >= 0.6
    sel = pos | (iou <= 0.45)
    arg = jnp.where(pos, p, 1.0 - p)
    l = jnp.maximum(jnp.log(arg), -100.0)
    s = jnp.sum(jnp.where(sel, -l, 0.0))
    c = jnp.sum(jnp.where(sel, 1.0, 0.0))

    @pl.when(step == 0)
    def _():
        acc_ref[0] = 0.0
        acc_ref[1] = 0.0

    acc_ref[0] += s
    acc_ref[1] += c

    @pl.when(step == pl.num_programs(0) - 1)
    def _():
        tot = acc_ref[0]
        cnt = acc_ref[1]
        out_ref[0, 0] = jnp.where(cnt > 0.0, tot / cnt, 0.0)


@jax.jit
def kernel(pred, iou):
    p2 = pred.reshape(_ROWS, _COLS)
    i2 = iou.reshape(_ROWS, _COLS)
    out = pl.pallas_call(
        _tc_body,
        grid=(_ROWS // _BLK_ROWS,),
        in_specs=[
            pl.BlockSpec((_BLK_ROWS, _COLS), lambda i: (i, 0)),
            pl.BlockSpec((_BLK_ROWS, _COLS), lambda i: (i, 0)),
        ],
        out_specs=pl.BlockSpec(
            (1, 1), lambda i: (0, 0), memory_space=pltpu.SMEM
        ),
        out_shape=jax.ShapeDtypeStruct((1, 1), jnp.float32),
        scratch_shapes=[pltpu.SMEM((2,), jnp.float32)],
    )(p2, i2)
    return out[0, 0]
